# baseline (speedup 1.0000x reference)
import functools

import jax
import jax.numpy as jnp
from jax import lax
from jax.experimental import pallas as pl
from jax.experimental.pallas import tpu as pltpu
from jax.experimental.pallas import tpu_sc as plsc

_BVS = 1000000
_BD = 64
_MD = 1024
_B, _S = 4, 2048
_N = _B * _S
_NC, _NS, _L = 2, 16, 16
_NW = _NC * _NS
_CHUNK = _N // _NW
_PAD = 8
_CH = 16384
_STEPS = (_BVS + _CH - 1) // _CH
_QROWS = _STEPS * _CH // 4
_SH = _CH.bit_length() - 1
_QS = _CH // 4


def _tc_relayout(table_t):
    def body(in_ref, o_ref):
        r = lax.broadcasted_iota(jnp.int32, (_BD, _BD), 0)
        c = lax.broadcasted_iota(jnp.int32, (_BD, _BD), 1)
        eye = (r == c).astype(jnp.bfloat16)
        t = lax.dot_general(in_ref[...].astype(jnp.bfloat16), eye,
                            (((0,), (0,)), ((), ())),
                            preferred_element_type=jnp.float32)
        u = lax.bitcast_convert_type(t, jnp.int32)
        q = _CH // 4
        p01 = lax.shift_right_logical(u[:q], 16) | u[q:2 * q]
        p23 = lax.shift_right_logical(u[2 * q:3 * q], 16) | u[3 * q:]
        o_ref[...] = jnp.concatenate([p01, p23], axis=1)

    return pl.pallas_call(
        body,
        grid=(_STEPS,),
        in_specs=[pl.BlockSpec((_BD, _CH), lambda i: (0, i))],
        out_specs=pl.BlockSpec((_CH // 4, 2 * _BD), lambda i: (i, 0)),
        out_shape=jax.ShapeDtypeStruct((_QROWS, 2 * _BD), jnp.int32),
    )(table_t)


def _sc_hash_gather(ids_pad, table4):
    mesh = plsc.VectorSubcoreMesh(core_axis_name="c", subcore_axis_name="s")

    @functools.partial(
        pl.kernel,
        mesh=mesh,
        out_type=(
            jax.ShapeDtypeStruct((_N, 128), jnp.int32),
            jax.ShapeDtypeStruct((_N,), jnp.float32),
        ),
        scratch_types=[
            pltpu.VMEM((_CHUNK + _PAD,), jnp.int32),
            pltpu.VMEM((2, 128), jnp.int32),
            pltpu.VMEM((_CHUNK,), jnp.float32),
            pltpu.VMEM((_CHUNK, 128), jnp.int32),
            pltpu.SemaphoreType.DMA,
        ],
    )
    def run(ids_hbm, table_hbm, out_hbm, quar_hbm,
            buf_v, idx_v, quar_v, rows_v, sem):
        wid = lax.axis_index("s") * _NC + lax.axis_index("c")
        base = wid * _CHUNK
        pltpu.sync_copy(ids_hbm.at[pl.ds(base, _CHUNK + _PAD)], buf_v)

        lanes = lax.iota(jnp.int32, _L)
        for i in range(_CHUNK // _L):
            off = _PAD + i * _L
            cur = buf_v[pl.ds(off, _L)]
            prev = buf_v[pl.ds(off - 1, _L)]
            h = lax.rem(jnp.bitwise_xor(cur * 36313, prev * 27191),
                        jnp.int32(_BVS - 1))
            pos = base + i * _L + lanes
            h = jnp.where((pos & (_S - 1)) == 0, jnp.int32(_BVS - 1), h)
            idx_v[i // 8, pl.ds((i % 8) * _L, _L)] = (
                ((h >> _SH) << (_SH - 2)) + (h & (_QS - 1)))
            quar_v[pl.ds(i * _L, _L)] = (
                (h >> (_SH - 2)) & 3).astype(jnp.float32)

        cps = [
            pltpu.async_copy(table_hbm.at[idx_v.at[r]],
                             rows_v.at[pl.ds(r * 128, 128)], sem)
            for r in range(2)
        ]
        for cp in cps:
            cp.wait()
        pltpu.sync_copy(rows_v, out_hbm.at[pl.ds(base, _CHUNK)])
        pltpu.sync_copy(quar_v, quar_hbm.at[pl.ds(base, _CHUNK)])

    return run(ids_pad, table4)


def _tc_project(blocks, quar, w, scale):
    blk = 1024

    def body(s_ref, b_ref, q_ref, w_ref, o_ref):
        wds = b_ref[...]
        q = q_ref[...]
        halves = [wds[:, :_BD], wds[:, _BD:]]
        rows = jnp.zeros((blk, _BD), jnp.float32)
        for k in range(4):
            bits = halves[k // 2]
            if k % 2 == 0:
                bits = lax.shift_left(bits, 16)
            else:
                bits = bits & jnp.int32(-65536)
            val = lax.bitcast_convert_type(bits, jnp.float32)
            rows = rows + jnp.where(q == float(k), val, 0.0)
        acc = lax.dot_general(rows, w_ref[...],
                              (((1,), (1,)), ((), ())),
                              preferred_element_type=jnp.float32)
        o_ref[...] = acc * s_ref[0, 0]

    return pl.pallas_call(
        body,
        grid=(_N // blk,),
        in_specs=[
            pl.BlockSpec(memory_space=pltpu.SMEM),
            pl.BlockSpec((blk, 128), lambda i: (i, 0)),
            pl.BlockSpec((blk, 1), lambda i: (i, 0)),
            pl.BlockSpec((_MD, _BD), lambda i: (0, 0)),
        ],
        out_specs=pl.BlockSpec((blk, _MD), lambda i: (i, 0)),
        out_shape=jax.ShapeDtypeStruct((_N, _MD), jnp.float32),
    )(scale, blocks, quar, w)


def kernel(ids, embed_weight, proj_weight, scale):
    ids_flat = ids.astype(jnp.int32).reshape(_N)
    ids_pad = jnp.concatenate([jnp.zeros((_PAD,), jnp.int32), ids_flat])
    table4 = _tc_relayout(embed_weight.T)
    blocks, quar = _sc_hash_gather(ids_pad, table4)
    out = _tc_project(blocks, quar.reshape(_N, 1), proj_weight,
                      scale.astype(jnp.float32).reshape(1, 1))
    return out.reshape(_B, _S, _MD)


# speedup vs baseline: 1.1064x; 1.1064x over previous
import functools

import jax
import jax.numpy as jnp
from jax import lax
from jax.experimental import pallas as pl
from jax.experimental.pallas import tpu as pltpu
from jax.experimental.pallas import tpu_sc as plsc

_BVS = 1000000
_BD = 64
_MD = 1024
_B, _S = 4, 2048
_N = _B * _S
_NC, _NS, _L = 2, 16, 16
_NW = _NC * _NS
_CHUNK = _N // _NW
_PAD = 8
_CH = 32768
_STEPS = (_BVS + _CH - 1) // _CH
_QROWS = _STEPS * _CH // 4
_SH = _CH.bit_length() - 1
_QS = _CH // 4


def _tc_relayout(table_t):
    def body(in_ref, o_ref):
        r = lax.broadcasted_iota(jnp.int32, (_BD, _BD), 0)
        c = lax.broadcasted_iota(jnp.int32, (_BD, _BD), 1)
        eye = (r == c).astype(jnp.bfloat16)
        t = lax.dot_general(in_ref[...].astype(jnp.bfloat16), eye,
                            (((0,), (0,)), ((), ())),
                            preferred_element_type=jnp.float32)
        u = lax.bitcast_convert_type(t, jnp.int32)
        q = _CH // 4
        p01 = lax.shift_right_logical(u[:q], 16) | u[q:2 * q]
        p23 = lax.shift_right_logical(u[2 * q:3 * q], 16) | u[3 * q:]
        o_ref[...] = jnp.concatenate([p01, p23], axis=1)

    return pl.pallas_call(
        body,
        grid=(_STEPS,),
        in_specs=[pl.BlockSpec((_BD, _CH), lambda i: (0, i))],
        out_specs=pl.BlockSpec((_CH // 4, 2 * _BD), lambda i: (i, 0)),
        out_shape=jax.ShapeDtypeStruct((_QROWS, 2 * _BD), jnp.int32),
    )(table_t)


def _sc_hash_gather(ids_pad, table4):
    mesh = plsc.VectorSubcoreMesh(core_axis_name="c", subcore_axis_name="s")

    @functools.partial(
        pl.kernel,
        mesh=mesh,
        out_type=(
            jax.ShapeDtypeStruct((_N, 128), jnp.int32),
            jax.ShapeDtypeStruct((_N,), jnp.float32),
        ),
        scratch_types=[
            pltpu.VMEM((_CHUNK + _PAD,), jnp.int32),
            pltpu.VMEM((2, 128), jnp.int32),
            pltpu.VMEM((_CHUNK,), jnp.float32),
            pltpu.VMEM((_CHUNK, 128), jnp.int32),
            pltpu.SemaphoreType.DMA,
        ],
    )
    def run(ids_hbm, table_hbm, out_hbm, quar_hbm,
            buf_v, idx_v, quar_v, rows_v, sem):
        wid = lax.axis_index("s") * _NC + lax.axis_index("c")
        base = wid * _CHUNK
        pltpu.sync_copy(ids_hbm.at[pl.ds(base, _CHUNK + _PAD)], buf_v)

        lanes = lax.iota(jnp.int32, _L)
        for i in range(_CHUNK // _L):
            off = _PAD + i * _L
            cur = buf_v[pl.ds(off, _L)]
            prev = buf_v[pl.ds(off - 1, _L)]
            h = lax.rem(jnp.bitwise_xor(cur * 36313, prev * 27191),
                        jnp.int32(_BVS - 1))
            pos = base + i * _L + lanes
            h = jnp.where((pos & (_S - 1)) == 0, jnp.int32(_BVS - 1), h)
            idx_v[i // 8, pl.ds((i % 8) * _L, _L)] = (
                ((h >> _SH) << (_SH - 2)) + (h & (_QS - 1)))
            quar_v[pl.ds(i * _L, _L)] = (
                (h >> (_SH - 2)) & 3).astype(jnp.float32)

        cps = [
            pltpu.async_copy(table_hbm.at[idx_v.at[r]],
                             rows_v.at[pl.ds(r * 128, 128)], sem)
            for r in range(2)
        ]
        for cp in cps:
            cp.wait()
        pltpu.sync_copy(rows_v, out_hbm.at[pl.ds(base, _CHUNK)])
        pltpu.sync_copy(quar_v, quar_hbm.at[pl.ds(base, _CHUNK)])

    return run(ids_pad, table4)


def _tc_project(blocks, quar, w, scale):
    blk = 1024

    def body(s_ref, b_ref, q_ref, w_ref, o_ref):
        wds = b_ref[...]
        q = q_ref[...]
        halves = [wds[:, :_BD], wds[:, _BD:]]
        rows = jnp.zeros((blk, _BD), jnp.float32)
        for k in range(4):
            bits = halves[k // 2]
            if k % 2 == 0:
                bits = lax.shift_left(bits, 16)
            else:
                bits = bits & jnp.int32(-65536)
            val = lax.bitcast_convert_type(bits, jnp.float32)
            rows = rows + jnp.where(q == float(k), val, 0.0)
        acc = lax.dot_general(rows, w_ref[...],
                              (((1,), (1,)), ((), ())),
                              preferred_element_type=jnp.float32)
        o_ref[...] = acc * s_ref[0, 0]

    return pl.pallas_call(
        body,
        grid=(_N // blk,),
        in_specs=[
            pl.BlockSpec(memory_space=pltpu.SMEM),
            pl.BlockSpec((blk, 128), lambda i: (i, 0)),
            pl.BlockSpec((blk, 1), lambda i: (i, 0)),
            pl.BlockSpec((_MD, _BD), lambda i: (0, 0)),
        ],
        out_specs=pl.BlockSpec((blk, _MD), lambda i: (i, 0)),
        out_shape=jax.ShapeDtypeStruct((_N, _MD), jnp.float32),
    )(scale, blocks, quar, w)


def kernel(ids, embed_weight, proj_weight, scale):
    ids_flat = ids.astype(jnp.int32).reshape(_N)
    ids_pad = jnp.concatenate([jnp.zeros((_PAD,), jnp.int32), ids_flat])
    table4 = _tc_relayout(embed_weight.T)
    blocks, quar = _sc_hash_gather(ids_pad, table4)
    out = _tc_project(blocks, quar.reshape(_N, 1), proj_weight,
                      scale.astype(jnp.float32).reshape(1, 1))
    return out.reshape(_B, _S, _MD)


# speedup vs baseline: 1.1151x; 1.0078x over previous
import functools

import jax
import jax.numpy as jnp
from jax import lax
from jax.experimental import pallas as pl
from jax.experimental.pallas import tpu as pltpu
from jax.experimental.pallas import tpu_sc as plsc

_BVS = 1000000
_BD = 64
_MD = 1024
_B, _S = 4, 2048
_N = _B * _S
_NC, _NS, _L = 2, 16, 16
_NW = _NC * _NS
_CHUNK = _N // _NW
_PAD = 8
_CH = 32768
_STEPS = (_BVS + _CH - 1) // _CH
_QROWS = _STEPS * _CH // 4
_SH = _CH.bit_length() - 1
_QS = _CH // 4


def _tc_relayout(table_t):
    def body(in_ref, o_ref):
        r = lax.broadcasted_iota(jnp.int32, (_BD, _BD), 0)
        c = lax.broadcasted_iota(jnp.int32, (_BD, _BD), 1)
        eye = (r == c).astype(jnp.bfloat16)
        t = lax.dot_general(in_ref[...].astype(jnp.bfloat16), eye,
                            (((0,), (0,)), ((), ())),
                            preferred_element_type=jnp.float32)
        u = lax.bitcast_convert_type(t, jnp.int32)
        q = _CH // 4
        p01 = lax.shift_right_logical(u[:q], 16) | u[q:2 * q]
        p23 = lax.shift_right_logical(u[2 * q:3 * q], 16) | u[3 * q:]
        o_ref[...] = jnp.concatenate([p01, p23], axis=1)

    return pl.pallas_call(
        body,
        grid=(_STEPS,),
        in_specs=[pl.BlockSpec((_BD, _CH), lambda i: (0, i))],
        out_specs=pl.BlockSpec((_CH // 4, 2 * _BD), lambda i: (i, 0)),
        out_shape=jax.ShapeDtypeStruct((_QROWS, 2 * _BD), jnp.int32),
    )(table_t)


def _sc_hash_gather(ids_pad, table4):
    mesh = plsc.VectorSubcoreMesh(core_axis_name="c", subcore_axis_name="s")

    @functools.partial(
        pl.kernel,
        mesh=mesh,
        out_type=(
            jax.ShapeDtypeStruct((_N, 128), jnp.int32),
            jax.ShapeDtypeStruct((_N,), jnp.float32),
        ),
        scratch_types=[
            pltpu.VMEM((_CHUNK + _PAD,), jnp.int32),
            pltpu.VMEM((2, 128), jnp.int32),
            pltpu.VMEM((_CHUNK,), jnp.float32),
            pltpu.VMEM((_CHUNK, 128), jnp.int32),
            pltpu.SemaphoreType.DMA,
        ],
    )
    def run(ids_hbm, table_hbm, out_hbm, quar_hbm,
            buf_v, idx_v, quar_v, rows_v, sem):
        wid = lax.axis_index("s") * _NC + lax.axis_index("c")
        base = wid * _CHUNK
        pltpu.sync_copy(ids_hbm.at[pl.ds(base, _CHUNK + _PAD)], buf_v)

        lanes = lax.iota(jnp.int32, _L)
        for i in range(_CHUNK // _L):
            off = _PAD + i * _L
            cur = buf_v[pl.ds(off, _L)]
            prev = buf_v[pl.ds(off - 1, _L)]
            h = lax.rem(jnp.bitwise_xor(cur * 36313, prev * 27191),
                        jnp.int32(_BVS - 1))
            pos = base + i * _L + lanes
            h = jnp.where((pos & (_S - 1)) == 0, jnp.int32(_BVS - 1), h)
            idx_v[i // 8, pl.ds((i % 8) * _L, _L)] = (
                ((h >> _SH) << (_SH - 2)) + (h & (_QS - 1)))
            quar_v[pl.ds(i * _L, _L)] = (
                (h >> (_SH - 2)) & 3).astype(jnp.float32)

        cps = [
            pltpu.async_copy(table_hbm.at[idx_v.at[r]],
                             rows_v.at[pl.ds(r * 128, 128)], sem)
            for r in range(2)
        ]
        for cp in cps:
            cp.wait()
        pltpu.sync_copy(rows_v, out_hbm.at[pl.ds(base, _CHUNK)])
        pltpu.sync_copy(quar_v, quar_hbm.at[pl.ds(base, _CHUNK)])

    return run(ids_pad, table4)


def _tc_project(blocks, quar, w, scale):
    blk = 2048

    def body(s_ref, b_ref, q_ref, w_ref, o_ref):
        wds = b_ref[...]
        q = q_ref[...]
        halves = [wds[:, :_BD], wds[:, _BD:]]
        rows = jnp.zeros((blk, _BD), jnp.float32)
        for k in range(4):
            bits = halves[k // 2]
            if k % 2 == 0:
                bits = lax.shift_left(bits, 16)
            else:
                bits = bits & jnp.int32(-65536)
            val = lax.bitcast_convert_type(bits, jnp.float32)
            rows = rows + jnp.where(q == float(k), val, 0.0)
        acc = lax.dot_general(rows, w_ref[...],
                              (((1,), (1,)), ((), ())),
                              preferred_element_type=jnp.float32)
        o_ref[...] = acc * s_ref[0, 0]

    return pl.pallas_call(
        body,
        grid=(_N // blk,),
        in_specs=[
            pl.BlockSpec(memory_space=pltpu.SMEM),
            pl.BlockSpec((blk, 128), lambda i: (i, 0)),
            pl.BlockSpec((blk, 1), lambda i: (i, 0)),
            pl.BlockSpec((_MD, _BD), lambda i: (0, 0)),
        ],
        out_specs=pl.BlockSpec((blk, _MD), lambda i: (i, 0)),
        out_shape=jax.ShapeDtypeStruct((_N, _MD), jnp.float32),
    )(scale, blocks, quar, w)


def kernel(ids, embed_weight, proj_weight, scale):
    ids_flat = ids.astype(jnp.int32).reshape(_N)
    ids_pad = jnp.concatenate([jnp.zeros((_PAD,), jnp.int32), ids_flat])
    table4 = _tc_relayout(embed_weight.T)
    blocks, quar = _sc_hash_gather(ids_pad, table4)
    out = _tc_project(blocks, quar.reshape(_N, 1), proj_weight,
                      scale.astype(jnp.float32).reshape(1, 1))
    return out.reshape(_B, _S, _MD)
